# baseline (speedup 1.0000x reference)
import functools

import jax
import jax.numpy as jnp
from jax import lax
from jax.experimental import pallas as pl
from jax.experimental.pallas import tpu as pltpu
from jax.experimental.pallas import tpu_sc as plsc

D = 64
BB = 128
NBUF = 4
NT = 2
CH = 2048


def _linear_table(emb_t):
    v = emb_t.shape[1]
    g = (v + CH - 1) // CH
    def body(x_ref, i_ref, o_ref):
        xt = lax.dot_general(
            x_ref[...], i_ref[...], (((0,), (0,)), ((), ())),
            precision=lax.Precision.HIGHEST,
            preferred_element_type=jnp.float32,
        )
        o_ref[...] = jnp.concatenate([xt, xt], axis=1)

    return pl.pallas_call(
        body,
        grid=(g,),
        in_specs=[
            pl.BlockSpec((D, CH), lambda i: (0, i)),
            pl.BlockSpec((D, D), lambda i: (0, 0)),
        ],
        out_specs=pl.BlockSpec((CH, 2 * D), lambda i: (i, 0)),
        out_shape=jax.ShapeDtypeStruct((g * CH, 2 * D), jnp.float32),
    )(emb_t, jnp.eye(D, dtype=jnp.float32))


def _gather_phys(table_lin, idx_flat, batch, hist):
    info = plsc.get_sparse_core_info()
    nc, ns = info.num_cores, info.num_subcores
    nw = nc * ns
    nbt = batch // BB
    bt_per_w = nbt // nw
    blocks_per_w = bt_per_w * hist

    mesh = plsc.VectorSubcoreMesh(core_axis_name="c", subcore_axis_name="s")

    @functools.partial(
        pl.kernel,
        mesh=mesh,
        out_type=jax.ShapeDtypeStruct((hist, D // 8, nbt, 8, BB), jnp.float32),
        scratch_types=[
            pltpu.VMEM((BB * hist,), jnp.int32),
            pltpu.VMEM((bt_per_w, hist, BB), jnp.int32),
            pltpu.VMEM((NBUF, BB, 2 * D), jnp.float32),
            pltpu.VMEM((NT * D, BB + 1), jnp.float32),
            pltpu.SemaphoreType.DMA((NBUF,)),
            pltpu.SemaphoreType.DMA((NT,)),
            pltpu.SemaphoreType.DMA,
        ],
        compiler_params=pltpu.CompilerParams(
            use_tc_tiling_on_sc=False, needs_layout_passes=False
        ),
    )
    def k(table_hbm, idx_hbm, out_hbm, stage_v, idxt_v, gbuf, tbuf, gsem, osem, ssem):
        wid = lax.axis_index("s") * nc + lax.axis_index("c")
        bt0 = wid * bt_per_w
        iota = lax.iota(jnp.int32, 16)
        i_h = iota * hist
        drows = [[iota + (s * D + d0 * 16) for d0 in range(D // 16)] for s in range(NT)]

        for i in range(bt_per_w):
            pltpu.async_copy(
                idx_hbm.at[pl.ds((bt0 + i) * (BB * hist), BB * hist)], stage_v, ssem
            ).wait()

            @pl.loop(0, hist)
            def _(h, i=i):
                for b0 in range(BB // 16):
                    v = plsc.load_gather(stage_v, [i_h + (b0 * 16 * hist + h)])
                    idxt_v[i, h, pl.ds(b0 * 16, 16)] = v

        def fire_gather(j, slot):
            i, h = j // hist, j % hist
            pltpu.async_copy(
                table_hbm.at[idxt_v.at[i, h]], gbuf.at[slot], gsem.at[slot]
            )

        def wait_gather(j, slot):
            i, h = j // hist, j % hist
            pltpu.make_async_copy(
                table_hbm.at[idxt_v.at[i, h]], gbuf.at[slot], gsem.at[slot]
            ).wait()

        def fire_out(j, s):
            i, h = j // hist, j % hist
            for dt in range(D // 8):
                pltpu.async_copy(
                    tbuf.at[pl.ds(s * D + dt * 8, 8), pl.ds(0, BB)],
                    out_hbm.at[h, dt, bt0 + i],
                    osem.at[s],
                )

        def wait_out(j, s):
            i, h = j // hist, j % hist
            for dt in range(D // 8):
                pltpu.make_async_copy(
                    tbuf.at[pl.ds(s * D + dt * 8, 8), pl.ds(0, BB)],
                    out_hbm.at[h, dt, bt0 + i],
                    osem.at[s],
                ).wait()

        for j in range(NBUF):
            fire_gather(j, j)

        @pl.loop(0, blocks_per_w // NBUF)
        def _(g):
            j0 = g * NBUF
            for bi in range(NBUF):
                j = j0 + bi
                s = bi % NT
                wait_gather(j, bi)

                @pl.when(j >= NT)
                def _(j=j, s=s):
                    wait_out(j - NT, s)

                for b in range(BB):
                    cols = jnp.full((16,), b, jnp.int32)
                    for d0 in range(D // 16):
                        v = gbuf[bi, b, pl.ds(d0 * 16, 16)]
                        plsc.store_scatter(tbuf, [drows[s][d0], cols], v)

                fire_out(j, s)

                @pl.when(j + NBUF < blocks_per_w)
                def _(j=j, bi=bi):
                    fire_gather(j + NBUF, bi)

        for j in range(blocks_per_w - NT, blocks_per_w):
            wait_out(j, (j % NBUF) % NT)

    return k(table_lin, idx_flat)


@functools.partial(jax.jit, static_argnames=("batch", "hist"))
def _run(emb_item, idx_flat, batch, hist):
    dup = _linear_table(emb_item.T)
    out5 = _gather_phys(dup, idx_flat, batch, hist)
    return out5.transpose(2, 4, 0, 1, 3).reshape(batch, hist, D)


def kernel(item_id, emb_item):
    batch, hist = item_id.shape
    idx_flat = item_id.astype(jnp.int32).reshape(batch * hist)
    return _run(emb_item, idx_flat, batch=batch, hist=hist)


# speedup vs baseline: 1.1038x; 1.1038x over previous
import functools

import jax
import jax.numpy as jnp
from jax import lax
from jax.experimental import pallas as pl
from jax.experimental.pallas import tpu as pltpu
from jax.experimental.pallas import tpu_sc as plsc

D = 64
BB = 128
NBUF = 4
NT = 2
CH = 2048


def _linear_table(emb_t):
    v = emb_t.shape[1]
    g = (v + CH - 1) // CH
    def body(x_ref, i_ref, o_ref):
        xt = lax.dot_general(
            x_ref[...], i_ref[...], (((0,), (0,)), ((), ())),
            preferred_element_type=jnp.float32,
        )
        o_ref[...] = jnp.concatenate([xt, xt], axis=1)

    return pl.pallas_call(
        body,
        grid=(g,),
        in_specs=[
            pl.BlockSpec((D, CH), lambda i: (0, i)),
            pl.BlockSpec((D, D), lambda i: (0, 0)),
        ],
        out_specs=pl.BlockSpec((CH, 2 * D), lambda i: (i, 0)),
        out_shape=jax.ShapeDtypeStruct((g * CH, 2 * D), jnp.float32),
    )(emb_t, jnp.eye(D, dtype=jnp.float32))


def _gather_phys(table_lin, idx_flat, batch, hist):
    info = plsc.get_sparse_core_info()
    nc, ns = info.num_cores, info.num_subcores
    nw = nc * ns
    nbt = batch // BB
    bt_per_w = nbt // nw
    blocks_per_w = bt_per_w * hist

    mesh = plsc.VectorSubcoreMesh(core_axis_name="c", subcore_axis_name="s")

    @functools.partial(
        pl.kernel,
        mesh=mesh,
        out_type=jax.ShapeDtypeStruct((hist, D // 8, nbt, 8, BB), jnp.float32),
        scratch_types=[
            pltpu.VMEM((BB * hist,), jnp.int32),
            pltpu.VMEM((bt_per_w, hist, BB), jnp.int32),
            pltpu.VMEM((NBUF, BB, 2 * D), jnp.float32),
            pltpu.VMEM((NT * D, BB + 1), jnp.float32),
            pltpu.SemaphoreType.DMA((NBUF,)),
            pltpu.SemaphoreType.DMA((NT,)),
            pltpu.SemaphoreType.DMA,
        ],
        compiler_params=pltpu.CompilerParams(
            use_tc_tiling_on_sc=False, needs_layout_passes=False
        ),
    )
    def k(table_hbm, idx_hbm, out_hbm, stage_v, idxt_v, gbuf, tbuf, gsem, osem, ssem):
        wid = lax.axis_index("s") * nc + lax.axis_index("c")
        bt0 = wid * bt_per_w
        iota = lax.iota(jnp.int32, 16)
        i_h = iota * hist
        drows = [[iota + (s * D + d0 * 16) for d0 in range(D // 16)] for s in range(NT)]

        for i in range(bt_per_w):
            pltpu.async_copy(
                idx_hbm.at[pl.ds((bt0 + i) * (BB * hist), BB * hist)], stage_v, ssem
            ).wait()

            @pl.loop(0, hist)
            def _(h, i=i):
                for b0 in range(BB // 16):
                    v = plsc.load_gather(stage_v, [i_h + (b0 * 16 * hist + h)])
                    idxt_v[i, h, pl.ds(b0 * 16, 16)] = v

        def fire_gather(j, slot):
            i, h = j // hist, j % hist
            pltpu.async_copy(
                table_hbm.at[idxt_v.at[i, h]], gbuf.at[slot], gsem.at[slot]
            )

        def wait_gather(j, slot):
            i, h = j // hist, j % hist
            pltpu.make_async_copy(
                table_hbm.at[idxt_v.at[i, h]], gbuf.at[slot], gsem.at[slot]
            ).wait()

        def fire_out(j, s):
            i, h = j // hist, j % hist
            for dt in range(D // 8):
                pltpu.async_copy(
                    tbuf.at[pl.ds(s * D + dt * 8, 8), pl.ds(0, BB)],
                    out_hbm.at[h, dt, bt0 + i],
                    osem.at[s],
                )

        def wait_out(j, s):
            i, h = j // hist, j % hist
            for dt in range(D // 8):
                pltpu.make_async_copy(
                    tbuf.at[pl.ds(s * D + dt * 8, 8), pl.ds(0, BB)],
                    out_hbm.at[h, dt, bt0 + i],
                    osem.at[s],
                ).wait()

        for j in range(NBUF):
            fire_gather(j, j)

        @pl.loop(0, blocks_per_w // NBUF)
        def _(g):
            j0 = g * NBUF
            for bi in range(NBUF):
                j = j0 + bi
                s = bi % NT
                wait_gather(j, bi)

                @pl.when(j >= NT)
                def _(j=j, s=s):
                    wait_out(j - NT, s)

                for b in range(BB):
                    cols = jnp.full((16,), b, jnp.int32)
                    for d0 in range(D // 16):
                        v = gbuf[bi, b, pl.ds(d0 * 16, 16)]
                        plsc.store_scatter(tbuf, [drows[s][d0], cols], v)

                fire_out(j, s)

                @pl.when(j + NBUF < blocks_per_w)
                def _(j=j, bi=bi):
                    fire_gather(j + NBUF, bi)

        for j in range(blocks_per_w - NT, blocks_per_w):
            wait_out(j, (j % NBUF) % NT)

    return k(table_lin, idx_flat)


@functools.partial(jax.jit, static_argnames=("batch", "hist"))
def _run(emb_item, idx_flat, batch, hist):
    dup = _linear_table(emb_item.T)
    out5 = _gather_phys(dup, idx_flat, batch, hist)
    return out5.transpose(2, 4, 0, 1, 3).reshape(batch, hist, D)


def kernel(item_id, emb_item):
    batch, hist = item_id.shape
    idx_flat = item_id.astype(jnp.int32).reshape(batch * hist)
    return _run(emb_item, idx_flat, batch=batch, hist=hist)
